# baseline (speedup 1.0000x reference)
import functools

import jax
import jax.numpy as jnp
from jax import lax
from jax.experimental import pallas as pl
from jax.experimental.pallas import tpu as pltpu
from jax.experimental.pallas import tpu_sc as plsc

B, L, D = 16, 4096, 128
EV_SC = 1536
R_TC = L - EV_SC
NW = 32
EV_PER_W = B * EV_SC // NW
NCHUNK = D // 16
C = 512


def _prep_body(la_ref, lb_ref, mu_ref, tb_ref, tp_ref,
               aT_ref, bT_ref, base_ref, ndt_ref):
    alpha = jax.nn.softplus(la_ref[...])
    beta = jax.nn.softplus(lb_ref[...])
    aT_ref[...] = alpha.T
    bT_ref[...] = beta.T
    tcol = tb_ref[:, 0:1]
    mu_ = jax.nn.softplus(mu_ref[0, :])
    base_ref[...] = mu_[None, :] * tcol
    ndt_ref[...] = tp_ref[...] - tcol


def _sc_body(aT_hbm, bT_hbm, et_hbm, ndt_hbm, out_hbm,
             aT_v, bT_v, et_v, ndt_v, acc_v):
    cid = lax.axis_index("c")
    sid = lax.axis_index("s")
    wid = sid * 2 + cid
    pltpu.sync_copy(aT_hbm, aT_v)
    pltpu.sync_copy(bT_hbm, bT_v)
    pltpu.sync_copy(et_hbm.at[wid], et_v)
    pltpu.sync_copy(ndt_hbm.at[wid], ndt_v)

    zeros = tuple(jnp.zeros((16,), jnp.float32) for _ in range(NCHUNK))

    @plsc.parallel_loop(0, EV_PER_W, 16, carry=(zeros, zeros))
    def _loop(j, carry):
        s1, s2 = carry
        etv = et_v[pl.ds(j, 16)]
        ndv = ndt_v[pl.ds(j, 16)]
        s1, s2 = list(s1), list(s2)
        for i in range(16):
            e = etv[i]
            nd = ndv[i]
            for k in range(NCHUNK):
                av = aT_v[e, pl.ds(k * 16, 16)]
                bv = bT_v[e, pl.ds(k * 16, 16)]
                s1[k] = s1[k] + av
                s2[k] = s2[k] + av * jnp.exp(bv * nd)
        return tuple(s1), tuple(s2)

    s1, s2 = _loop
    for k in range(NCHUNK):
        acc_v[pl.ds(k * 16, 16)] = s1[k] - s2[k]
    pltpu.sync_copy(acc_v, out_hbm.at[wid])


def _tc_body(tp_ref, et_ref, tb_ref, a_ref, b_ref, out_ref):
    lidx = pl.program_id(1)
    tp = tp_ref[0, 0, 0, :]
    et = et_ref[0, 0, 0, :]
    tb = tb_ref[0, 0, 0]
    alpha = a_ref[...]
    beta = b_ref[...]

    onehot = (et[:, None] == lax.broadcasted_iota(jnp.int32, (C, D), 1)
              ).astype(jnp.float32)
    dn = (((1,), (0,)), ((), ()))
    rel_a = lax.dot_general(onehot, alpha, dn,
                            preferred_element_type=jnp.float32)
    rel_b = lax.dot_general(onehot, beta, dn,
                            preferred_element_type=jnp.float32)
    ndt = (tp - tb)[:, None]
    decayed = rel_a * jnp.exp(rel_b * ndt)
    ones = jnp.full((8, C), 1.0, dtype=jnp.float32)
    dn2 = (((1,), (0,)), ((), ()))
    partial = lax.dot_general(ones, rel_a - decayed, dn2,
                              preferred_element_type=jnp.float32)[0:1]

    @pl.when(lidx == 0)
    def _init():
        out_ref[0, :, :] = partial

    @pl.when(lidx != 0)
    def _acc():
        out_ref[0, :, :] += partial


def kernel(T, time_points, event_types, mu, log_alpha, log_beta):
    tb = jnp.broadcast_to(T[:, None], (B, 128))
    mu2 = mu.reshape(1, D)

    aT, bT, base, ndt = pl.pallas_call(
        _prep_body,
        out_shape=[
            jax.ShapeDtypeStruct((D, D), jnp.float32),
            jax.ShapeDtypeStruct((D, D), jnp.float32),
            jax.ShapeDtypeStruct((B, D), jnp.float32),
            jax.ShapeDtypeStruct((B, L), jnp.float32),
        ],
    )(log_alpha, log_beta, mu2, tb, time_points)

    et_w = event_types[:, :EV_SC].reshape(NW, EV_PER_W)
    ndt_w = ndt[:, :EV_SC].reshape(NW, EV_PER_W)
    sc = pl.kernel(
        _sc_body,
        out_type=jax.ShapeDtypeStruct((NW, D), jnp.float32),
        mesh=plsc.VectorSubcoreMesh(core_axis_name="c", subcore_axis_name="s"),
        scratch_types=[
            pltpu.VMEM((D, D), jnp.float32),
            pltpu.VMEM((D, D), jnp.float32),
            pltpu.VMEM((EV_PER_W,), jnp.int32),
            pltpu.VMEM((EV_PER_W,), jnp.float32),
            pltpu.VMEM((D,), jnp.float32),
        ],
    )
    part_sc = sc(aT, bT, et_w, ndt_w)

    tp3 = time_points[:, EV_SC:].reshape(B, R_TC // C, 1, C)
    et3 = event_types[:, EV_SC:].reshape(B, R_TC // C, 1, C)
    tb3 = tb.reshape(B, 1, 128)
    part_tc = pl.pallas_call(
        _tc_body,
        grid=(B, R_TC // C),
        in_specs=[
            pl.BlockSpec((1, 1, 1, C), lambda b, l: (b, l, 0, 0)),
            pl.BlockSpec((1, 1, 1, C), lambda b, l: (b, l, 0, 0)),
            pl.BlockSpec((1, 1, 128), lambda b, l: (b, 0, 0)),
            pl.BlockSpec((D, D), lambda b, l: (0, 0)),
            pl.BlockSpec((D, D), lambda b, l: (0, 0)),
        ],
        out_specs=pl.BlockSpec((1, 1, D), lambda b, l: (b, 0, 0)),
        out_shape=jax.ShapeDtypeStruct((B, 1, D), jnp.float32),
    )(tp3, et3, tb3, aT, bT)

    return base + part_sc.reshape(B, 2, D).sum(axis=1) + part_tc.reshape(B, D)


# speedup vs baseline: 1.5768x; 1.5768x over previous
import functools

import jax
import jax.numpy as jnp
from jax import lax
from jax.experimental import pallas as pl
from jax.experimental.pallas import tpu as pltpu
from jax.experimental.pallas import tpu_sc as plsc

B, L, D = 16, 4096, 128
EV_SC = 2048
R_TC = L - EV_SC
NW = 32
EV_PER_W = B * EV_SC // NW
NCHUNK = D // 16
C = 1024


def _prep_body(la_ref, lb_ref, mu_ref, tb_ref, tp_ref,
               aT_ref, bT_ref, base_ref, ndt_ref):
    alpha = jax.nn.softplus(la_ref[...])
    beta = jax.nn.softplus(lb_ref[...])
    aT_ref[...] = alpha.T
    bT_ref[...] = beta.T
    tcol = tb_ref[:, 0:1]
    mu_ = jax.nn.softplus(mu_ref[0, :])
    base_ref[...] = mu_[None, :] * tcol
    ndt_ref[...] = tp_ref[...] - tcol


def _sc_body(aT_hbm, bT_hbm, et_hbm, ndt_hbm, out_hbm,
             aT_v, bT_v, et_v, ndt_v, acc_v):
    cid = lax.axis_index("c")
    sid = lax.axis_index("s")
    wid = sid * 2 + cid
    pltpu.sync_copy(aT_hbm, aT_v)
    pltpu.sync_copy(bT_hbm, bT_v)
    pltpu.sync_copy(et_hbm.at[wid], et_v)
    pltpu.sync_copy(ndt_hbm.at[wid], ndt_v)

    zeros = tuple(jnp.zeros((16,), jnp.float32) for _ in range(NCHUNK))

    @plsc.parallel_loop(0, EV_PER_W, 16, carry=(zeros, zeros))
    def _loop(j, carry):
        s1, s2 = carry
        etv = et_v[pl.ds(j, 16)]
        ndv = ndt_v[pl.ds(j, 16)]
        s1, s2 = list(s1), list(s2)
        for i in range(16):
            e = etv[i]
            nd = ndv[i]
            for k in range(NCHUNK):
                av = aT_v[e, pl.ds(k * 16, 16)]
                bv = bT_v[e, pl.ds(k * 16, 16)]
                s1[k] = s1[k] + av
                s2[k] = s2[k] + av * jnp.exp(bv * nd)
        return tuple(s1), tuple(s2)

    s1, s2 = _loop
    for k in range(NCHUNK):
        acc_v[pl.ds(k * 16, 16)] = s1[k] - s2[k]
    pltpu.sync_copy(acc_v, out_hbm.at[wid])


def _tc_body(tp_ref, et_ref, tb_ref, a_ref, b_ref, out_ref):
    lidx = pl.program_id(1)
    tp = tp_ref[0, 0, 0, :]
    et = et_ref[0, 0, 0, :]
    tb = tb_ref[0, 0, 0]
    alpha = a_ref[...]
    beta = b_ref[...]

    onehot = (et[:, None] == lax.broadcasted_iota(jnp.int32, (C, D), 1)
              ).astype(jnp.float32)
    dn = (((1,), (0,)), ((), ()))
    rel_a = lax.dot_general(onehot, alpha, dn,
                            preferred_element_type=jnp.float32)
    rel_b = lax.dot_general(onehot, beta, dn,
                            preferred_element_type=jnp.float32)
    ndt = (tp - tb)[:, None]
    decayed = rel_a * jnp.exp(rel_b * ndt)
    ones = jnp.full((8, C), 1.0, dtype=jnp.float32)
    dn2 = (((1,), (0,)), ((), ()))
    partial = lax.dot_general(ones, rel_a - decayed, dn2,
                              preferred_element_type=jnp.float32)[0:1]

    @pl.when(lidx == 0)
    def _init():
        out_ref[0, :, :] = partial

    @pl.when(lidx != 0)
    def _acc():
        out_ref[0, :, :] += partial


def kernel(T, time_points, event_types, mu, log_alpha, log_beta):
    tb = jnp.broadcast_to(T[:, None], (B, 128))
    mu2 = mu.reshape(1, D)

    aT, bT, base, ndt = pl.pallas_call(
        _prep_body,
        out_shape=[
            jax.ShapeDtypeStruct((D, D), jnp.float32),
            jax.ShapeDtypeStruct((D, D), jnp.float32),
            jax.ShapeDtypeStruct((B, D), jnp.float32),
            jax.ShapeDtypeStruct((B, L), jnp.float32),
        ],
    )(log_alpha, log_beta, mu2, tb, time_points)

    et_w = event_types[:, :EV_SC].reshape(NW, EV_PER_W)
    ndt_w = ndt[:, :EV_SC].reshape(NW, EV_PER_W)
    sc = pl.kernel(
        _sc_body,
        out_type=jax.ShapeDtypeStruct((NW, D), jnp.float32),
        mesh=plsc.VectorSubcoreMesh(core_axis_name="c", subcore_axis_name="s"),
        scratch_types=[
            pltpu.VMEM((D, D), jnp.float32),
            pltpu.VMEM((D, D), jnp.float32),
            pltpu.VMEM((EV_PER_W,), jnp.int32),
            pltpu.VMEM((EV_PER_W,), jnp.float32),
            pltpu.VMEM((D,), jnp.float32),
        ],
    )
    part_sc = sc(aT, bT, et_w, ndt_w)

    tp3 = time_points[:, EV_SC:].reshape(B, R_TC // C, 1, C)
    et3 = event_types[:, EV_SC:].reshape(B, R_TC // C, 1, C)
    tb3 = tb.reshape(B, 1, 128)
    part_tc = pl.pallas_call(
        _tc_body,
        grid=(B, R_TC // C),
        in_specs=[
            pl.BlockSpec((1, 1, 1, C), lambda b, l: (b, l, 0, 0)),
            pl.BlockSpec((1, 1, 1, C), lambda b, l: (b, l, 0, 0)),
            pl.BlockSpec((1, 1, 128), lambda b, l: (b, 0, 0)),
            pl.BlockSpec((D, D), lambda b, l: (0, 0)),
            pl.BlockSpec((D, D), lambda b, l: (0, 0)),
        ],
        out_specs=pl.BlockSpec((1, 1, D), lambda b, l: (b, 0, 0)),
        out_shape=jax.ShapeDtypeStruct((B, 1, D), jnp.float32),
    )(tp3, et3, tb3, aT, bT)

    return base + part_sc.reshape(B, 2, D).sum(axis=1) + part_tc.reshape(B, D)
